# baseline (speedup 1.0000x reference)
import functools
import math

import jax
import jax.numpy as jnp
from jax import lax
from jax.experimental import pallas as pl
from jax.experimental.pallas import tpu as pltpu
from jax.experimental.pallas import tpu_sc as plsc

BN = 16384
B = 256
V = 100000
D_ATOM = 64
D_T = 64
HALF_T = D_T // 2
D_FF = 128
H = 128
D_OUT = D_ATOM + D_T + D_FF

ROWS = 2048
NBLK = BN // ROWS


def _make_sc_gather():
    nc, ns = 2, 16
    nw = nc * ns
    b_per_w = BN // nw
    mesh = plsc.VectorSubcoreMesh(core_axis_name="c", subcore_axis_name="s")

    @functools.partial(
        pl.kernel,
        mesh=mesh,
        out_type=jax.ShapeDtypeStruct((BN, 128), jnp.float32),
        scratch_types=[
            pltpu.VMEM((b_per_w,), jnp.int32),
            pltpu.VMEM((b_per_w, 128), jnp.float32),
            pltpu.SemaphoreType.DMA,
        ],
        compiler_params=pltpu.CompilerParams(use_tc_tiling_on_sc=False),
    )
    def gather_k(table2_hbm, idx_hbm, out_hbm, idx_v, rows_v, sem):
        wid = lax.axis_index("s") * nc + lax.axis_index("c")
        base = wid * b_per_w
        pltpu.sync_copy(idx_hbm.at[pl.ds(base, b_per_w)], idx_v)

        def mod_body(k, _):
            a = idx_v[pl.ds(k * 16, 16)]
            idx_v[pl.ds(k * 16, 16)] = jnp.where(a >= V_PAD, a - V_PAD, a)
            return _

        lax.fori_loop(0, b_per_w // 16, mod_body, 0)
        pltpu.async_copy(table2_hbm.at[idx_v], rows_v, sem).wait()
        pltpu.sync_copy(rows_v, out_hbm.at[pl.ds(base, b_per_w)])

    return gather_k


RP_COLS = 512
RP_GRID = 98
V_PAD = RP_COLS * RP_GRID


def _repack_body(ttl_ref, ttr_ref, out_ref):
    blk = jnp.concatenate([ttl_ref[...], ttr_ref[...]], axis=0)
    out_ref[...] = jnp.transpose(blk)


def _repack(tt, interpret=False):
    return pl.pallas_call(
        _repack_body,
        grid=(RP_GRID,),
        in_specs=[
            pl.BlockSpec((D_ATOM, RP_COLS), lambda i: (0, i)),
            pl.BlockSpec((D_ATOM, RP_COLS), lambda i: (0, i + RP_GRID)),
        ],
        out_specs=pl.BlockSpec((RP_COLS, 128), lambda i: (i, 0)),
        out_shape=jax.ShapeDtypeStruct((V_PAD, 128), jnp.float32),
        interpret=interpret,
    )(tt, tt)


_sc_gather_cache = []


def _sc_gather(table, idx):
    if not _sc_gather_cache:
        _sc_gather_cache.append(_make_sc_gather())
    return _sc_gather_cache[0](table, idx)


def _tc_body(t_ref, bid_ref, ffin_ref, atp_ref, atom_ref,
             w1_ref, b1_ref, w2_ref, b2_ref, w3_ref, b3_ref, out_ref, temb_ref):
    @pl.when(pl.program_id(0) == 0)
    def _compute_temb():
        ji = lax.broadcasted_iota(jnp.int32, (D_T, B), 0)
        j = ji.astype(jnp.float32)
        jh = jnp.where(j >= HALF_T, j - HALF_T, j)
        freqs = jnp.exp(jh * (-math.log(10000.0) / HALF_T))
        args = t_ref[...] * freqs
        temb_ref[...] = jnp.where(ji < HALF_T, jnp.sin(args), jnp.cos(args))

    tembT = temb_ref[...]

    ohT = (lax.broadcasted_iota(jnp.int32, (B, ROWS), 0) == bid_ref[...]).astype(jnp.float32)
    t_full = lax.dot_general(ohT, tembT, (((0,), (1,)), ((), ())),
                             preferred_element_type=jnp.float32)

    x = ffin_ref[...]
    h = jnp.maximum(lax.dot_general(x, w1_ref[...], (((0,), (0,)), ((), ())),
                                    preferred_element_type=jnp.float32) + b1_ref[...], 0.0)
    h = jnp.maximum(jnp.dot(h, w2_ref[...], preferred_element_type=jnp.float32) + b2_ref[...], 0.0)
    ff = jnp.dot(h, w3_ref[...], preferred_element_type=jnp.float32) + b3_ref[...]

    par_col = jnp.transpose((atp_ref[...] >= V_PAD).astype(jnp.int32))
    g = atom_ref[...]
    atom = jnp.where(par_col != 0, g[:, D_ATOM:], g[:, :D_ATOM])
    out_ref[...] = jnp.concatenate([atom, t_full, ff], axis=-1)


def _tc_call(t_row, bid_row, ffin, atp_row, atom_emb, w1, b1, w2, b2, w3, b3,
             interpret=False):
    return pl.pallas_call(
        _tc_body,
        grid=(NBLK,),
        in_specs=[
            pl.BlockSpec((1, B), lambda i: (0, 0)),
            pl.BlockSpec((1, ROWS), lambda i: (0, i)),
            pl.BlockSpec((4, ROWS), lambda i: (0, i)),
            pl.BlockSpec((1, ROWS), lambda i: (0, i)),
            pl.BlockSpec((ROWS, 128), lambda i: (i, 0)),
            pl.BlockSpec((4, H), lambda i: (0, 0)),
            pl.BlockSpec((1, H), lambda i: (0, 0)),
            pl.BlockSpec((H, H), lambda i: (0, 0)),
            pl.BlockSpec((1, H), lambda i: (0, 0)),
            pl.BlockSpec((H, D_FF), lambda i: (0, 0)),
            pl.BlockSpec((1, D_FF), lambda i: (0, 0)),
        ],
        out_specs=pl.BlockSpec((ROWS, D_OUT), lambda i: (i, 0)),
        out_shape=jax.ShapeDtypeStruct((BN, D_OUT), jnp.float32),
        scratch_shapes=[pltpu.VMEM((D_T, B), jnp.float32)],
        interpret=interpret,
    )(t_row, bid_row, ffin, atp_row, atom_emb, w1, b1, w2, b2, w3, b3)


def kernel(atom_type, t_interpolant, batch_ids, charge, mass, sigma, epsilon,
           atom_table, W1, b1, W2, b2, W3, b3):
    table2 = _repack(atom_table.T)
    atom_emb = _sc_gather(table2, atom_type.astype(jnp.int32))
    t_row = t_interpolant.astype(jnp.float32).reshape(1, B)
    bid_row = batch_ids.astype(jnp.int32).reshape(1, BN)
    atp_row = atom_type.astype(jnp.int32).reshape(1, BN)
    ffin = jnp.concatenate(
        [charge.astype(jnp.float32).T, mass.astype(jnp.float32).T,
         sigma.astype(jnp.float32).T, epsilon.astype(jnp.float32).T], axis=0)
    return _tc_call(
        t_row, bid_row, ffin, atp_row, atom_emb,
        W1, b1.reshape(1, H), W2, b2.reshape(1, H), W3, b3.reshape(1, D_FF),
    )


# speedup vs baseline: 1.2943x; 1.2943x over previous
import functools
import math

import jax
import jax.numpy as jnp
from jax import lax
from jax.experimental import pallas as pl
from jax.experimental.pallas import tpu as pltpu
from jax.experimental.pallas import tpu_sc as plsc

BN = 16384
B = 256
V = 100000
D_ATOM = 64
D_T = 64
HALF_T = D_T // 2
D_FF = 128
H = 128
D_OUT = D_ATOM + D_T + D_FF

ROWS = 2048
NBLK = BN // ROWS


def _make_sc_gather():
    nc, ns = 2, 16
    nw = nc * ns
    b_per_w = BN // nw
    mesh = plsc.VectorSubcoreMesh(core_axis_name="c", subcore_axis_name="s")

    @functools.partial(
        pl.kernel,
        mesh=mesh,
        out_type=jax.ShapeDtypeStruct((BN, 128), jnp.float32),
        scratch_types=[
            pltpu.VMEM((b_per_w,), jnp.int32),
            pltpu.VMEM((b_per_w, 128), jnp.float32),
            pltpu.SemaphoreType.DMA,
        ],
        compiler_params=pltpu.CompilerParams(use_tc_tiling_on_sc=False),
    )
    def gather_k(table2_hbm, idx_hbm, out_hbm, idx_v, rows_v, sem):
        wid = lax.axis_index("s") * nc + lax.axis_index("c")
        base = wid * b_per_w
        pltpu.sync_copy(idx_hbm.at[pl.ds(base, b_per_w)], idx_v)

        def mod_body(k, _):
            a = idx_v[pl.ds(k * 16, 16)]
            idx_v[pl.ds(k * 16, 16)] = jnp.where(a >= V_PAD, a - V_PAD, a)
            return _

        lax.fori_loop(0, b_per_w // 16, mod_body, 0)
        pltpu.async_copy(table2_hbm.at[idx_v], rows_v, sem).wait()
        pltpu.sync_copy(rows_v, out_hbm.at[pl.ds(base, b_per_w)])

    return gather_k


RP_COLS = 1024
RP_GRID = 49
V_PAD = RP_COLS * RP_GRID


def _repack_body(ttl_ref, ttr_ref, out_ref):
    blk = jnp.concatenate([ttl_ref[...], ttr_ref[...]], axis=0)
    out_ref[...] = jnp.transpose(blk)


def _repack(tt, interpret=False):
    return pl.pallas_call(
        _repack_body,
        grid=(RP_GRID,),
        in_specs=[
            pl.BlockSpec((D_ATOM, RP_COLS), lambda i: (0, i)),
            pl.BlockSpec((D_ATOM, RP_COLS), lambda i: (0, i + RP_GRID)),
        ],
        out_specs=pl.BlockSpec((RP_COLS, 128), lambda i: (i, 0)),
        out_shape=jax.ShapeDtypeStruct((V_PAD, 128), jnp.float32),
        interpret=interpret,
    )(tt, tt)


_sc_gather_cache = []


def _sc_gather(table, idx):
    if not _sc_gather_cache:
        _sc_gather_cache.append(_make_sc_gather())
    return _sc_gather_cache[0](table, idx)


def _tc_body(t_ref, bid_ref, ffin_ref, atp_ref, atom_ref,
             w1_ref, b1_ref, w2_ref, b2_ref, w3_ref, b3_ref, out_ref, temb_ref):
    @pl.when(pl.program_id(0) == 0)
    def _compute_temb():
        ji = lax.broadcasted_iota(jnp.int32, (D_T, B), 0)
        j = ji.astype(jnp.float32)
        jh = jnp.where(j >= HALF_T, j - HALF_T, j)
        freqs = jnp.exp(jh * (-math.log(10000.0) / HALF_T))
        args = t_ref[...] * freqs
        temb_ref[...] = jnp.where(ji < HALF_T, jnp.sin(args), jnp.cos(args))

    tembT = temb_ref[...]

    ohT = (lax.broadcasted_iota(jnp.int32, (B, ROWS), 0) == bid_ref[...]).astype(jnp.float32)
    t_full = lax.dot_general(ohT, tembT, (((0,), (1,)), ((), ())),
                             preferred_element_type=jnp.float32)

    x = ffin_ref[...]
    h = jnp.maximum(lax.dot_general(x, w1_ref[...], (((0,), (0,)), ((), ())),
                                    preferred_element_type=jnp.float32) + b1_ref[...], 0.0)
    h = jnp.maximum(jnp.dot(h, w2_ref[...], preferred_element_type=jnp.float32) + b2_ref[...], 0.0)
    ff = jnp.dot(h, w3_ref[...], preferred_element_type=jnp.float32) + b3_ref[...]

    par_col = jnp.transpose((atp_ref[...] >= V_PAD).astype(jnp.int32))
    g = atom_ref[...]
    atom = jnp.where(par_col != 0, g[:, D_ATOM:], g[:, :D_ATOM])
    out_ref[...] = jnp.concatenate([atom, t_full, ff], axis=-1)


def _tc_call(t_row, bid_row, ffin, atp_row, atom_emb, w1, b1, w2, b2, w3, b3,
             interpret=False):
    return pl.pallas_call(
        _tc_body,
        grid=(NBLK,),
        in_specs=[
            pl.BlockSpec((1, B), lambda i: (0, 0)),
            pl.BlockSpec((1, ROWS), lambda i: (0, i)),
            pl.BlockSpec((4, ROWS), lambda i: (0, i)),
            pl.BlockSpec((1, ROWS), lambda i: (0, i)),
            pl.BlockSpec((ROWS, 128), lambda i: (i, 0)),
            pl.BlockSpec((4, H), lambda i: (0, 0)),
            pl.BlockSpec((1, H), lambda i: (0, 0)),
            pl.BlockSpec((H, H), lambda i: (0, 0)),
            pl.BlockSpec((1, H), lambda i: (0, 0)),
            pl.BlockSpec((H, D_FF), lambda i: (0, 0)),
            pl.BlockSpec((1, D_FF), lambda i: (0, 0)),
        ],
        out_specs=pl.BlockSpec((ROWS, D_OUT), lambda i: (i, 0)),
        out_shape=jax.ShapeDtypeStruct((BN, D_OUT), jnp.float32),
        scratch_shapes=[pltpu.VMEM((D_T, B), jnp.float32)],
        interpret=interpret,
    )(t_row, bid_row, ffin, atp_row, atom_emb, w1, b1, w2, b2, w3, b3)


def kernel(atom_type, t_interpolant, batch_ids, charge, mass, sigma, epsilon,
           atom_table, W1, b1, W2, b2, W3, b3):
    table2 = _repack(atom_table.T)
    atom_emb = _sc_gather(table2, atom_type.astype(jnp.int32))
    t_row = t_interpolant.astype(jnp.float32).reshape(1, B)
    bid_row = batch_ids.astype(jnp.int32).reshape(1, BN)
    atp_row = atom_type.astype(jnp.int32).reshape(1, BN)
    ffin = jnp.concatenate(
        [charge.astype(jnp.float32).T, mass.astype(jnp.float32).T,
         sigma.astype(jnp.float32).T, epsilon.astype(jnp.float32).T], axis=0)
    return _tc_call(
        t_row, bid_row, ffin, atp_row, atom_emb,
        W1, b1.reshape(1, H), W2, b2.reshape(1, H), W3, b3.reshape(1, D_FF),
    )
